# baseline (speedup 1.0000x reference)
import functools

import jax
import jax.numpy as jnp
from jax import lax
from jax.experimental import pallas as pl
from jax.experimental.pallas import tpu as pltpu
from jax.experimental.pallas import tpu_sc as plsc

_NC = 2
_NS = 16
_NW = _NC * _NS
_LANES = 16

_CH = 100
_G = 20
_NBUF = 2
_DCH = 100
_DG = 20


def _sc_mesh():
    return plsc.VectorSubcoreMesh(
        core_axis_name="c", subcore_axis_name="s",
        num_cores=_NC, num_subcores=_NS)


@functools.lru_cache(maxsize=None)
def _make_deg_kernel(n, ngroup):
    nper = n // _NS
    zr = 128
    assert nper % zr == 0

    @functools.partial(
        pl.kernel, mesh=_sc_mesh(),
        out_type=jax.ShapeDtypeStruct((_NC, n, _LANES), jnp.float32),
        scratch_types=[
            pltpu.VMEM((_DG, _DCH), jnp.int32),
            pltpu.VMEM((_DCH, _LANES), jnp.float32),
            pltpu.VMEM((zr, _LANES), jnp.float32),
            pltpu.VMEM_SHARED((n, _LANES), jnp.float32),
        ],
    )
    def deg_k(dst_hbm, out_hbm, idx_d, ones_v, zero_v, acc):
        c = lax.axis_index("c")
        s = lax.axis_index("s")
        wid = s * _NC + c
        zeros16 = jnp.zeros((_LANES,), jnp.float32)
        ones16 = jnp.ones((_LANES,), jnp.float32)

        @pl.loop(0, _DCH)
        def _(i):
            ones_v[i, :] = ones16

        @pl.loop(0, zr)
        def _(i):
            zero_v[i, :] = zeros16

        base = s * nper

        @pl.loop(0, nper // zr)
        def _(j):
            pltpu.sync_copy(zero_v, acc.at[pl.ds(base + j * zr, zr), :])

        plsc.subcore_barrier()

        @pl.loop(0, ngroup)
        def _(g):
            pltpu.sync_copy(dst_hbm.at[wid, g], idx_d)

            @pl.loop(0, _DG)
            def _(i):
                pltpu.sync_copy(ones_v, acc.at[idx_d.at[i]], add=True)

        plsc.subcore_barrier()

        @pl.loop(0, nper // zr)
        def _(j):
            r0 = base + j * zr
            pltpu.sync_copy(acc.at[pl.ds(r0, zr), :],
                            out_hbm.at[c, pl.ds(r0, zr), :])

    return deg_k


@functools.lru_cache(maxsize=None)
def _make_agg_kernel(n, f, ngroup):
    nper = n // _NS
    dr = 128
    assert nper % dr == 0
    assert f % _LANES == 0 and _G % _NBUF == 0

    @functools.partial(
        pl.kernel, mesh=_sc_mesh(),
        out_type=jax.ShapeDtypeStruct((_NC, n, f), jnp.float32),
        scratch_types=[
            pltpu.VMEM((_G, _CH), jnp.int32),
            pltpu.VMEM((_G, _CH), jnp.int32),
            pltpu.VMEM((_NBUF, _CH, f), jnp.float32),
            pltpu.VMEM((32, f), jnp.float32),
            pltpu.SemaphoreType.DMA((_NBUF,)),
            pltpu.SemaphoreType.DMA((_NBUF,)),
            pltpu.VMEM_SHARED((n, f), jnp.float32),
        ],
    )
    def agg_k(g_hbm, src_hbm, dst_hbm, out_hbm, idx_s, idx_d, rows, zero_v,
              sem_g, sem_s, acc):
        c = lax.axis_index("c")
        s = lax.axis_index("s")
        wid = s * _NC + c
        zeros16 = jnp.zeros((_LANES,), jnp.float32)

        def buf(j):
            return rows.at[j]

        zc = 32

        @pl.loop(0, zc)
        def _(i):
            @pl.loop(0, f // _LANES)
            def _(j):
                zero_v[i, pl.ds(j * _LANES, _LANES)] = zeros16

        base = s * nper

        @pl.loop(0, nper // zc)
        def _(j):
            pltpu.sync_copy(zero_v, acc.at[pl.ds(base + j * zc, zc), :])

        plsc.subcore_barrier()

        @pl.loop(0, ngroup)
        def _(g):
            pltpu.sync_copy(src_hbm.at[wid, g], idx_s)
            pltpu.sync_copy(dst_hbm.at[wid, g], idx_d)
            for j in range(_NBUF):
                pltpu.async_copy(g_hbm.at[idx_s.at[j]], buf(j), sem_g.at[j])

            @pl.loop(0, _G, step=_NBUF)
            def _(k):
                for j in range(_NBUF):
                    i = k + j
                    pltpu.make_async_copy(g_hbm.at[idx_s.at[i]], buf(j),
                                          sem_g.at[j]).wait()
                    pltpu.sync_copy(buf(j), acc.at[idx_d.at[i]], add=True)

                    @pl.when(i + _NBUF < _G)
                    def _():
                        pltpu.async_copy(g_hbm.at[idx_s.at[i + _NBUF]],
                                         buf(j), sem_g.at[j])

        plsc.subcore_barrier()

        @pl.loop(0, nper // dr)
        def _(j):
            r0 = base + j * dr
            pltpu.sync_copy(acc.at[pl.ds(r0, dr), :],
                            out_hbm.at[c, pl.ds(r0, dr), :])

    return agg_k


def _tc_layer1(cnt, x, w1):
    n = x.shape[0]
    h_dim = w1.shape[1]

    def body(cnt_ref, x_ref, w1_ref, g_ref, dis_ref):
        count = cnt_ref[0][:, 0:1] + cnt_ref[1][:, 0:1]
        dis = lax.rsqrt(count + 1.0)
        h = jnp.dot(x_ref[...], w1_ref[...],
                    preferred_element_type=jnp.float32)
        g_ref[...] = h * dis
        dis_ref[...] = dis

    return pl.pallas_call(
        body,
        out_shape=(jax.ShapeDtypeStruct((n, h_dim), jnp.float32),
                   jax.ShapeDtypeStruct((n, 1), jnp.float32)),
    )(cnt, x, w1)


def _tc_mid(aggp, g1, dis, b1, w2, fpad):
    n = g1.shape[0]
    c_dim = w2.shape[1]

    def body(aggp_ref, g1_ref, dis_ref, b1_ref, w2_ref, g2_ref):
        agg = aggp_ref[0] + aggp_ref[1] + g1_ref[...]
        dis = dis_ref[...]
        out1 = jnp.maximum(agg * dis + b1_ref[...], 0.0)
        h2 = jnp.dot(out1, w2_ref[...], preferred_element_type=jnp.float32)
        g2 = h2 * dis
        g2_ref[...] = jnp.concatenate(
            [g2, jnp.zeros((n, fpad - c_dim), jnp.float32)], axis=1)

    return pl.pallas_call(
        body,
        out_shape=jax.ShapeDtypeStruct((n, fpad), jnp.float32),
    )(aggp, g1, dis, b1, w2)


def _tc_final(aggp, g2, dis, b2, c_dim):
    n = g2.shape[0]

    def body(aggp_ref, g2_ref, dis_ref, b2_ref, o_ref):
        agg = (aggp_ref[0] + aggp_ref[1] + g2_ref[...])[:, :c_dim]
        o = agg * dis_ref[...] + b2_ref[...]
        m = jnp.max(o, axis=1, keepdims=True)
        e = jnp.exp(o - m)
        lse = jnp.log(jnp.sum(e, axis=1, keepdims=True)) + m
        o_ref[...] = o - lse

    return pl.pallas_call(
        body,
        out_shape=jax.ShapeDtypeStruct((n, c_dim), jnp.float32),
    )(aggp, g2, dis, b2)


def kernel(x, edge_index, W1, b1, W2, b2):
    n, d = x.shape
    e = edge_index.shape[1]
    h_dim = W1.shape[1]
    c_dim = W2.shape[1]

    epw = e // _NW
    ngroup = epw // (_G * _CH)
    ngroup_d = epw // (_DG * _DCH)
    assert e == _NW * ngroup * _G * _CH == _NW * ngroup_d * _DG * _DCH

    src = edge_index[0].reshape(_NW, ngroup, _G, _CH)
    dst = edge_index[1].reshape(_NW, ngroup, _G, _CH)
    dst_d = edge_index[1].reshape(_NW, ngroup_d, _DG, _DCH)

    npad = -(-n // (_NS * 128)) * (_NS * 128)

    fpad = 128

    cnt = _make_deg_kernel(npad, ngroup_d)(dst_d)[:, :n]
    g1, dis = _tc_layer1(cnt, x, W1)
    p1 = _make_agg_kernel(npad, h_dim, ngroup)(g1, src, dst)[:, :n]
    g2 = _tc_mid(p1, g1, dis, b1.reshape(1, h_dim), W2, fpad)
    p2 = _make_agg_kernel(npad, fpad, ngroup)(g2, src, dst)[:, :n]
    return _tc_final(p2, g2, dis, b2.reshape(1, c_dim), c_dim)


# speedup vs baseline: 1.1516x; 1.1516x over previous
import functools

import jax
import jax.numpy as jnp
from jax import lax
from jax.experimental import pallas as pl
from jax.experimental.pallas import tpu as pltpu
from jax.experimental.pallas import tpu_sc as plsc

_NC = 2
_NS = 16
_NW = _NC * _NS
_LANES = 16

_CH = 50
_G = 40
_NBUF = 4
_DCH = 125
_DG = 16


def _sc_mesh():
    return plsc.VectorSubcoreMesh(
        core_axis_name="c", subcore_axis_name="s",
        num_cores=_NC, num_subcores=_NS)


@functools.lru_cache(maxsize=None)
def _make_deg_kernel(n, ngroup):
    nper = n // _NS
    zr = 128
    assert nper % zr == 0

    @functools.partial(
        pl.kernel, mesh=_sc_mesh(),
        out_type=jax.ShapeDtypeStruct((_NC, n, _LANES), jnp.float32),
        scratch_types=[
            pltpu.VMEM((_DG, _DCH), jnp.int32),
            pltpu.VMEM((_DCH, _LANES), jnp.float32),
            pltpu.VMEM((zr, _LANES), jnp.float32),
            pltpu.VMEM_SHARED((n, _LANES), jnp.float32),
        ],
    )
    def deg_k(dst_hbm, out_hbm, idx_d, ones_v, zero_v, acc):
        c = lax.axis_index("c")
        s = lax.axis_index("s")
        wid = s * _NC + c
        zeros16 = jnp.zeros((_LANES,), jnp.float32)
        ones16 = jnp.ones((_LANES,), jnp.float32)

        @pl.loop(0, _DCH)
        def _(i):
            ones_v[i, :] = ones16

        @pl.loop(0, zr)
        def _(i):
            zero_v[i, :] = zeros16

        base = s * nper

        @pl.loop(0, nper // zr)
        def _(j):
            pltpu.sync_copy(zero_v, acc.at[pl.ds(base + j * zr, zr), :])

        plsc.subcore_barrier()

        @pl.loop(0, ngroup)
        def _(g):
            pltpu.sync_copy(dst_hbm.at[wid, g], idx_d)

            @pl.loop(0, _DG)
            def _(i):
                pltpu.sync_copy(ones_v, acc.at[idx_d.at[i]], add=True)

        plsc.subcore_barrier()

        @pl.loop(0, nper // zr)
        def _(j):
            r0 = base + j * zr
            pltpu.sync_copy(acc.at[pl.ds(r0, zr), :],
                            out_hbm.at[c, pl.ds(r0, zr), :])

    return deg_k


@functools.lru_cache(maxsize=None)
def _make_agg_kernel(n, f, ngroup):
    nper = n // _NS
    dr = 128
    assert nper % dr == 0
    assert f % _LANES == 0 and _G % _NBUF == 0

    @functools.partial(
        pl.kernel, mesh=_sc_mesh(),
        out_type=jax.ShapeDtypeStruct((_NC, n, f), jnp.float32),
        scratch_types=[
            pltpu.VMEM((_G, _CH), jnp.int32),
            pltpu.VMEM((_G, _CH), jnp.int32),
            pltpu.VMEM((_NBUF, _CH, f), jnp.float32),
            pltpu.VMEM((32, f), jnp.float32),
            pltpu.SemaphoreType.DMA((_NBUF,)),
            pltpu.SemaphoreType.DMA((_NBUF,)),
            pltpu.VMEM_SHARED((n, f), jnp.float32),
        ],
    )
    def agg_k(g_hbm, src_hbm, dst_hbm, out_hbm, idx_s, idx_d, rows, zero_v,
              sem_g, sem_s, acc):
        c = lax.axis_index("c")
        s = lax.axis_index("s")
        wid = s * _NC + c
        zeros16 = jnp.zeros((_LANES,), jnp.float32)

        def buf(j):
            return rows.at[j]

        zc = 32

        @pl.loop(0, zc)
        def _(i):
            @pl.loop(0, f // _LANES)
            def _(j):
                zero_v[i, pl.ds(j * _LANES, _LANES)] = zeros16

        base = s * nper

        @pl.loop(0, nper // zc)
        def _(j):
            pltpu.sync_copy(zero_v, acc.at[pl.ds(base + j * zc, zc), :])

        plsc.subcore_barrier()

        @pl.loop(0, ngroup)
        def _(g):
            pltpu.sync_copy(src_hbm.at[wid, g], idx_s)
            pltpu.sync_copy(dst_hbm.at[wid, g], idx_d)
            for j in range(_NBUF):
                pltpu.async_copy(g_hbm.at[idx_s.at[j]], buf(j), sem_g.at[j])

            @pl.loop(0, _G, step=_NBUF)
            def _(k):
                for j in range(_NBUF):
                    i = k + j
                    pltpu.make_async_copy(g_hbm.at[idx_s.at[i]], buf(j),
                                          sem_g.at[j]).wait()
                    pltpu.sync_copy(buf(j), acc.at[idx_d.at[i]], add=True)

                    @pl.when(i + _NBUF < _G)
                    def _():
                        pltpu.async_copy(g_hbm.at[idx_s.at[i + _NBUF]],
                                         buf(j), sem_g.at[j])

        plsc.subcore_barrier()

        @pl.loop(0, nper // dr)
        def _(j):
            r0 = base + j * dr
            pltpu.sync_copy(acc.at[pl.ds(r0, dr), :],
                            out_hbm.at[c, pl.ds(r0, dr), :])

    return agg_k


def _dis_from_cnt(cnt_ref, n):
    count = cnt_ref[0][:n, 0:1] + cnt_ref[1][:n, 0:1]
    return lax.rsqrt(count + 1.0)


def _tc_layer1(cnt, x, w1):
    n = x.shape[0]
    h_dim = w1.shape[1]

    def body(cnt_ref, x_ref, w1_ref, g_ref):
        dis = _dis_from_cnt(cnt_ref, n)
        h = jnp.dot(x_ref[...], w1_ref[...],
                    preferred_element_type=jnp.float32)
        g_ref[...] = h * dis

    return pl.pallas_call(
        body,
        out_shape=jax.ShapeDtypeStruct((n, h_dim), jnp.float32),
    )(cnt, x, w1)


def _tc_mid(aggp, g1, cnt, b1, w2, fpad):
    n = g1.shape[0]
    c_dim = w2.shape[1]

    def body(aggp_ref, g1_ref, cnt_ref, b1_ref, w2_ref, g2_ref):
        dis = _dis_from_cnt(cnt_ref, n)
        agg = aggp_ref[0][:n] + aggp_ref[1][:n] + g1_ref[...]
        out1 = jnp.maximum(agg * dis + b1_ref[...], 0.0)
        h2 = jnp.dot(out1, w2_ref[...], preferred_element_type=jnp.float32)
        g2 = h2 * dis
        g2_ref[...] = jnp.concatenate(
            [g2, jnp.zeros((n, fpad - c_dim), jnp.float32)], axis=1)

    return pl.pallas_call(
        body,
        out_shape=jax.ShapeDtypeStruct((n, fpad), jnp.float32),
    )(aggp, g1, cnt, b1, w2)


def _tc_final(aggp, g2, cnt, b2, c_dim):
    n = g2.shape[0]

    def body(aggp_ref, g2_ref, cnt_ref, b2_ref, o_ref):
        dis = _dis_from_cnt(cnt_ref, n)
        agg = (aggp_ref[0][:n] + aggp_ref[1][:n] + g2_ref[...])[:, :c_dim]
        o = agg * dis + b2_ref[...]
        m = jnp.max(o, axis=1, keepdims=True)
        e = jnp.exp(o - m)
        lse = jnp.log(jnp.sum(e, axis=1, keepdims=True)) + m
        o_ref[...] = o - lse

    return pl.pallas_call(
        body,
        out_shape=jax.ShapeDtypeStruct((n, c_dim), jnp.float32),
    )(aggp, g2, cnt, b2)


def kernel(x, edge_index, W1, b1, W2, b2):
    n, d = x.shape
    e = edge_index.shape[1]
    h_dim = W1.shape[1]
    c_dim = W2.shape[1]

    epw = e // _NW
    ngroup = epw // (_G * _CH)
    ngroup_d = epw // (_DG * _DCH)
    assert e == _NW * ngroup * _G * _CH == _NW * ngroup_d * _DG * _DCH

    src = edge_index[0].reshape(_NW, ngroup, _G, _CH)
    dst = edge_index[1].reshape(_NW, ngroup, _G, _CH)
    dst_d = edge_index[1].reshape(_NW, ngroup_d, _DG, _DCH)

    npad = -(-n // (_NS * 128)) * (_NS * 128)

    fpad = 128

    cnt = _make_deg_kernel(npad, ngroup_d)(dst_d)
    g1 = _tc_layer1(cnt, x, W1)
    p1 = _make_agg_kernel(npad, h_dim, ngroup)(g1, src, dst)
    g2 = _tc_mid(p1, g1, cnt, b1.reshape(1, h_dim), W2, fpad)
    p2 = _make_agg_kernel(npad, fpad, ngroup)(g2, src, dst)
    return _tc_final(p2, g2, cnt, b2.reshape(1, c_dim), c_dim)


# speedup vs baseline: 1.1546x; 1.0026x over previous
import functools

import jax
import jax.numpy as jnp
from jax import lax
from jax.experimental import pallas as pl
from jax.experimental.pallas import tpu as pltpu
from jax.experimental.pallas import tpu_sc as plsc

_NC = 2
_NS = 16
_NW = _NC * _NS
_LANES = 16

_CH = 50
_G = 40
_NBUF = 4
_DCH = 125
_DG = 16


def _sc_mesh():
    return plsc.VectorSubcoreMesh(
        core_axis_name="c", subcore_axis_name="s",
        num_cores=_NC, num_subcores=_NS)


@functools.lru_cache(maxsize=None)
def _make_deg_kernel(n, ngroup):
    nper = n // _NS
    zr = 128
    assert nper % zr == 0

    @functools.partial(
        pl.kernel, mesh=_sc_mesh(),
        out_type=jax.ShapeDtypeStruct((_NC, n, _LANES), jnp.float32),
        scratch_types=[
            pltpu.VMEM((_DG, _DCH), jnp.int32),
            pltpu.VMEM((_DCH, _LANES), jnp.float32),
            pltpu.VMEM((zr, _LANES), jnp.float32),
            pltpu.VMEM_SHARED((n, _LANES), jnp.float32),
        ],
    )
    def deg_k(dst_hbm, out_hbm, idx_d, ones_v, zero_v, acc):
        c = lax.axis_index("c")
        s = lax.axis_index("s")
        wid = s * _NC + c
        zeros16 = jnp.zeros((_LANES,), jnp.float32)
        ones16 = jnp.ones((_LANES,), jnp.float32)

        @pl.loop(0, _DCH)
        def _(i):
            ones_v[i, :] = ones16

        @pl.loop(0, zr)
        def _(i):
            zero_v[i, :] = zeros16

        base = s * nper

        @pl.loop(0, nper // zr)
        def _(j):
            pltpu.sync_copy(zero_v, acc.at[pl.ds(base + j * zr, zr), :])

        plsc.subcore_barrier()

        @pl.loop(0, ngroup)
        def _(g):
            pltpu.sync_copy(dst_hbm.at[wid, g], idx_d)

            @pl.loop(0, _DG)
            def _(i):
                pltpu.sync_copy(ones_v, acc.at[idx_d.at[i]], add=True)

        plsc.subcore_barrier()

        @pl.loop(0, nper // zr)
        def _(j):
            r0 = base + j * zr
            pltpu.sync_copy(acc.at[pl.ds(r0, zr), :],
                            out_hbm.at[c, pl.ds(r0, zr), :])

    return deg_k


@functools.lru_cache(maxsize=None)
def _make_agg_kernel(n, f, ngroup):
    nper = n // _NS
    dr = 128
    assert nper % dr == 0
    assert f % _LANES == 0 and _G % _NBUF == 0

    @functools.partial(
        pl.kernel, mesh=_sc_mesh(),
        out_type=jax.ShapeDtypeStruct((_NC, n, f), jnp.float32),
        scratch_types=[
            pltpu.VMEM((_G, _CH), jnp.int32),
            pltpu.VMEM((_G, _CH), jnp.int32),
            pltpu.VMEM((_NBUF, _CH, f), jnp.float32),
            pltpu.VMEM((32, f), jnp.float32),
            pltpu.SemaphoreType.DMA((_NBUF,)),
            pltpu.SemaphoreType.DMA((_NBUF,)),
            pltpu.VMEM_SHARED((n, f), jnp.float32),
        ],
    )
    def agg_k(g_hbm, src_hbm, dst_hbm, out_hbm, idx_s, idx_d, rows, zero_v,
              sem_g, sem_s, acc):
        c = lax.axis_index("c")
        s = lax.axis_index("s")
        wid = s * _NC + c
        zeros16 = jnp.zeros((_LANES,), jnp.float32)

        def buf(j):
            return rows.at[j]

        zc = 32

        @pl.loop(0, zc)
        def _(i):
            @pl.loop(0, f // _LANES)
            def _(j):
                zero_v[i, pl.ds(j * _LANES, _LANES)] = zeros16

        base = s * nper

        @pl.loop(0, nper // zc)
        def _(j):
            pltpu.sync_copy(zero_v, acc.at[pl.ds(base + j * zc, zc), :])

        plsc.subcore_barrier()

        @pl.loop(0, ngroup)
        def _(g):
            pltpu.sync_copy(src_hbm.at[wid, g], idx_s)
            pltpu.sync_copy(dst_hbm.at[wid, g], idx_d)
            for j in range(_NBUF):
                pltpu.async_copy(g_hbm.at[idx_s.at[j]], buf(j), sem_g.at[j])

            @pl.loop(0, _G, step=_NBUF)
            def _(k):
                for j in range(_NBUF):
                    i = k + j
                    pltpu.make_async_copy(g_hbm.at[idx_s.at[i]], buf(j),
                                          sem_g.at[j]).wait()
                    pltpu.sync_copy(buf(j), acc.at[idx_d.at[i]], add=True)

                    @pl.when(i + _NBUF < _G)
                    def _():
                        pltpu.async_copy(g_hbm.at[idx_s.at[i + _NBUF]],
                                         buf(j), sem_g.at[j])

        plsc.subcore_barrier()

        @pl.loop(0, nper // dr)
        def _(j):
            r0 = base + j * dr
            pltpu.sync_copy(acc.at[pl.ds(r0, dr), :],
                            out_hbm.at[c, pl.ds(r0, dr), :])

    return agg_k


def _dis_from_cnt(cnt_ref, n):
    count = cnt_ref[0][:n, 0:1] + cnt_ref[1][:n, 0:1]
    return lax.rsqrt(count + 1.0)


def _tc_mm1(x, w1):
    n = x.shape[0]
    h_dim = w1.shape[1]

    def body(x_ref, w1_ref, h_ref):
        h_ref[...] = jnp.dot(x_ref[...], w1_ref[...],
                             preferred_element_type=jnp.float32)

    return pl.pallas_call(
        body,
        out_shape=jax.ShapeDtypeStruct((n, h_dim), jnp.float32),
    )(x, w1)


def _tc_scale(h1, cnt):
    n, h_dim = h1.shape

    def body(h_ref, cnt_ref, g_ref):
        dis = _dis_from_cnt(cnt_ref, n)
        g_ref[...] = h_ref[...] * dis

    return pl.pallas_call(
        body,
        out_shape=jax.ShapeDtypeStruct((n, h_dim), jnp.float32),
    )(h1, cnt)


def _tc_mid(aggp, g1, cnt, b1, w2, fpad):
    n = g1.shape[0]
    c_dim = w2.shape[1]

    def body(aggp_ref, g1_ref, cnt_ref, b1_ref, w2_ref, g2_ref):
        dis = _dis_from_cnt(cnt_ref, n)
        agg = aggp_ref[0][:n] + aggp_ref[1][:n] + g1_ref[...]
        out1 = jnp.maximum(agg * dis + b1_ref[...], 0.0)
        h2 = jnp.dot(out1, w2_ref[...], preferred_element_type=jnp.float32)
        g2 = h2 * dis
        if fpad > c_dim:
            g2 = jnp.concatenate(
                [g2, jnp.zeros((n, fpad - c_dim), jnp.float32)], axis=1)
        g2_ref[...] = g2

    return pl.pallas_call(
        body,
        out_shape=jax.ShapeDtypeStruct((n, fpad), jnp.float32),
    )(aggp, g1, cnt, b1, w2)


def _tc_final(aggp, g2, cnt, b2, c_dim):
    n = g2.shape[0]

    def body(aggp_ref, g2_ref, cnt_ref, b2_ref, o_ref):
        dis = _dis_from_cnt(cnt_ref, n)
        agg = (aggp_ref[0][:n] + aggp_ref[1][:n] + g2_ref[...])[:, :c_dim]
        o = agg * dis + b2_ref[...]
        m = jnp.max(o, axis=1, keepdims=True)
        e = jnp.exp(o - m)
        lse = jnp.log(jnp.sum(e, axis=1, keepdims=True)) + m
        o_ref[...] = o - lse

    return pl.pallas_call(
        body,
        out_shape=jax.ShapeDtypeStruct((n, c_dim), jnp.float32),
    )(aggp, g2, cnt, b2)


def kernel(x, edge_index, W1, b1, W2, b2):
    n, d = x.shape
    e = edge_index.shape[1]
    h_dim = W1.shape[1]
    c_dim = W2.shape[1]

    epw = e // _NW
    ngroup = epw // (_G * _CH)
    ngroup_d = epw // (_DG * _DCH)
    assert e == _NW * ngroup * _G * _CH == _NW * ngroup_d * _DG * _DCH

    src = edge_index[0].reshape(_NW, ngroup, _G, _CH)
    dst = edge_index[1].reshape(_NW, ngroup, _G, _CH)
    dst_d = edge_index[1].reshape(_NW, ngroup_d, _DG, _DCH)

    npad = -(-n // (_NS * 128)) * (_NS * 128)

    fpad = 128

    cnt = _make_deg_kernel(npad, ngroup_d)(dst_d)
    h1 = _tc_mm1(x, W1)
    g1 = _tc_scale(h1, cnt)
    p1 = _make_agg_kernel(npad, h_dim, ngroup)(g1, src, dst)
    g2 = _tc_mid(p1, g1, cnt, b1.reshape(1, h_dim), W2, fpad)
    p2 = _make_agg_kernel(npad, fpad, ngroup)(g2, src, dst)
    return _tc_final(p2, g2, cnt, b2.reshape(1, c_dim), c_dim)
